# baseline (speedup 1.0000x reference)
import jax
import jax.numpy as jnp
from jax.experimental import pallas as pl
from jax.experimental.pallas import tpu as pltpu

_TOTAL = 4 * 4096 * 2048
_NSTRIPE = 8
_STRIPE = _TOTAL // _NSTRIPE


def _dma_copy(x_ref, o_ref, *sems):
    cps = [
        pltpu.make_async_copy(
            x_ref.at[pl.ds(i * _STRIPE, _STRIPE)],
            o_ref.at[pl.ds(i * _STRIPE, _STRIPE)],
            sems[i],
        )
        for i in range(_NSTRIPE)
    ]
    for cp in cps:
        cp.start()
    for cp in cps:
        cp.wait()


def kernel(x):
    b, s, d = x.shape
    out = pl.pallas_call(
        _dma_copy,
        in_specs=[pl.BlockSpec(memory_space=pl.ANY)],
        out_specs=pl.BlockSpec(memory_space=pl.ANY),
        scratch_shapes=[pltpu.SemaphoreType.DMA] * _NSTRIPE,
        out_shape=jax.ShapeDtypeStruct((_TOTAL,), x.dtype),
    )(x.reshape(-1))
    return out.reshape(b, s, d)


# speedup vs baseline: 51.9944x; 51.9944x over previous
import jax
import jax.numpy as jnp
from jax.experimental import pallas as pl
from jax.experimental.pallas import tpu as pltpu

_BLOCK_ROWS = 1024


def _copy_block(x_ref, o_ref):
    o_ref[...] = x_ref[...]


def kernel(x):
    b, s, d = x.shape
    x2 = x.reshape(b * s, d)
    out = pl.pallas_call(
        _copy_block,
        grid=(x2.shape[0] // _BLOCK_ROWS,),
        in_specs=[pl.BlockSpec((_BLOCK_ROWS, d), lambda i: (i, 0))],
        out_specs=pl.BlockSpec((_BLOCK_ROWS, d), lambda i: (i, 0)),
        out_shape=jax.ShapeDtypeStruct(x2.shape, x2.dtype),
    )(x2)
    return out.reshape(b, s, d)
